# baseline (speedup 1.0000x reference)
import jax
import jax.numpy as jnp
from jax import lax
from jax.experimental import pallas as pl
from jax.experimental.pallas import tpu as pltpu
from jax.experimental.pallas import tpu_sc as plsc

NC = 2
NS = 16
NW = NC * NS
L = 16

B, P, CH = 128, 20000, 2
P_TOTAL = B * P
P_W = P_TOTAL // NW
CHUNK = 8_000
NCHUNK = P_W // CHUNK
VSTEPS = CHUNK // L


def _partial_body(regr_hbm, gt_hbm, mask_hbm, part_hbm,
                  mk0, mk1, rr0, rr1, gg0, gg1, stage_v,
                  sm0, sm1, sr0, sr1, sg0, sg1):
    cid = lax.axis_index("c")
    sid = lax.axis_index("s")
    wid = cid * NS + sid

    mbase = wid * P_W
    fbase = mbase * CH

    half_iota = lax.iota(jnp.int32, L) >> 1

    mk = (mk0, mk1)
    rr = (rr0, rr1)
    gg = (gg0, gg1)
    sems = ((sm0, sr0, sg0), (sm1, sr1, sg1))

    def start(s, c):
        sm, sr, sg = sems[s]
        pltpu.async_copy(mask_hbm.at[pl.ds(mbase + c * CHUNK, CHUNK)], mk[s], sm)
        pltpu.async_copy(regr_hbm.at[pl.ds(fbase + c * 2 * CHUNK, 2 * CHUNK)], rr[s], sr)
        pltpu.async_copy(gt_hbm.at[pl.ds(fbase + c * 2 * CHUNK, 2 * CHUNK)], gg[s], sg)

    def wait(s):
        sm, sr, sg = sems[s]
        pltpu.make_async_copy(mask_hbm.at[pl.ds(0, CHUNK)], mk[s], sm).wait()
        pltpu.make_async_copy(regr_hbm.at[pl.ds(0, 2 * CHUNK)], rr[s], sr).wait()
        pltpu.make_async_copy(gt_hbm.at[pl.ds(0, 2 * CHUNK)], gg[s], sg).wait()

    zero = jnp.zeros((L,), jnp.float32)

    start(0, 0)
    start(1, 1)

    def pair_body(c2, carry):
        acc, cnt = carry
        for s in (0, 1):
            c = c2 * 2 + s
            wait(s)

            mk_s, rr_s, gg_s = mk[s], rr[s], gg[s]

            def vstep(i, carry2, mk_s=mk_s, rr_s=rr_s, gg_s=gg_s):
                acc2, cnt2 = carry2
                r0 = rr_s[pl.ds(i * 2 * L, L)]
                g0 = gg_s[pl.ds(i * 2 * L, L)]
                r1 = rr_s[pl.ds(i * 2 * L + L, L)]
                g1 = gg_s[pl.ds(i * 2 * L + L, L)]
                idx0 = half_iota + i * L
                idx1 = idx0 + (L // 2)
                m0 = plsc.load_gather(mk_s, [idx0]).astype(jnp.float32)
                m1 = plsc.load_gather(mk_s, [idx1]).astype(jnp.float32)
                d0 = jnp.abs(r0 - g0)
                d1 = jnp.abs(r1 - g1)
                acc2 = acc2 + d0 * m0 + d1 * m1
                cnt2 = cnt2 + m0 + m1
                return acc2, cnt2

            acc, cnt = lax.fori_loop(0, VSTEPS, vstep, (acc, cnt))

            @pl.when(c + 2 < NCHUNK)
            def _():
                start(s, c + 2)
        return acc, cnt

    acc, cnt = lax.fori_loop(0, NCHUNK // 2, pair_body, (zero, zero))

    stage_v[pl.ds(0, L)] = acc
    stage_v[pl.ds(L, L)] = cnt
    pltpu.sync_copy(stage_v, part_hbm.at[wid])


def _reduce_body(part_hbm, out_hbm, red_v, res_v):
    cid = lax.axis_index("c")
    sid = lax.axis_index("s")
    wid = cid * NS + sid

    @pl.when(wid == 0)
    def _():
        pltpu.sync_copy(part_hbm, red_v)
        a = jnp.zeros((L,), jnp.float32)
        k = jnp.zeros((L,), jnp.float32)
        for w in range(NW):
            a = a + red_v[w, pl.ds(0, L)]
            k = k + red_v[w, pl.ds(L, L)]
        s_loss = jnp.sum(a)
        s_cnt = jnp.sum(k)
        res = jnp.full((L,), s_loss, jnp.float32) / (jnp.full((L,), s_cnt, jnp.float32) + 1e-4)
        res_v[...] = res
        pltpu.sync_copy(res_v, out_hbm)


@jax.jit
def _masked_l1(regr_f, gt_f, mask_f):
    mesh = plsc.VectorSubcoreMesh(core_axis_name="c", subcore_axis_name="s")
    partials = pl.kernel(
        _partial_body,
        out_type=jax.ShapeDtypeStruct((NW, 2 * L), jnp.float32),
        mesh=mesh,
        compiler_params=pltpu.CompilerParams(needs_layout_passes=False),
        scratch_types=[
            pltpu.VMEM((CHUNK,), jnp.int32),
            pltpu.VMEM((CHUNK,), jnp.int32),
            pltpu.VMEM((2 * CHUNK,), jnp.float32),
            pltpu.VMEM((2 * CHUNK,), jnp.float32),
            pltpu.VMEM((2 * CHUNK,), jnp.float32),
            pltpu.VMEM((2 * CHUNK,), jnp.float32),
            pltpu.VMEM((2 * L,), jnp.float32),
            pltpu.SemaphoreType.DMA,
            pltpu.SemaphoreType.DMA,
            pltpu.SemaphoreType.DMA,
            pltpu.SemaphoreType.DMA,
            pltpu.SemaphoreType.DMA,
            pltpu.SemaphoreType.DMA,
        ],
    )(regr_f, gt_f, mask_f)

    out = pl.kernel(
        _reduce_body,
        out_type=jax.ShapeDtypeStruct((L,), jnp.float32),
        mesh=mesh,
        compiler_params=pltpu.CompilerParams(needs_layout_passes=False),
        scratch_types=[
            pltpu.VMEM((NW, 2 * L), jnp.float32),
            pltpu.VMEM((L,), jnp.float32),
        ],
    )(partials)
    return out


def kernel(regr, gt_regr, mask):
    out = _masked_l1(regr.reshape(-1), gt_regr.reshape(-1), mask.reshape(-1))
    return out[0]


# speedup vs baseline: 120.8787x; 120.8787x over previous
import jax
import jax.numpy as jnp
from jax import lax
from jax.experimental import pallas as pl
from jax.experimental.pallas import tpu as pltpu
from jax.experimental.pallas import tpu_sc as plsc

NC = 2
NS = 16
NW = NC * NS
L = 16

B, P, CHN = 128, 20000, 2
POS_W = P // NW
CHUNK_P = 25
NCHUNK = POS_W // CHUNK_P
MSK_C = CHUNK_P * B
VAL_C = CHUNK_P * B * CHN


def _partial_body(regr_hbm, gt_hbm, mask_hbm, part_hbm,
                  mk0, mk1, rr0, rr1, gg0, gg1, stage_v,
                  sm0, sm1, sr0, sr1, sg0, sg1):
    cid = lax.axis_index("c")
    sid = lax.axis_index("s")
    wid = cid * NS + sid

    mbase = wid * POS_W * B
    fbase = wid * POS_W * B * CHN

    mk = (mk0, mk1)
    rr = (rr0, rr1)
    gg = (gg0, gg1)
    sems = ((sm0, sr0, sg0), (sm1, sr1, sg1))

    def start(s, c):
        sm, sr, sg = sems[s]
        pltpu.async_copy(mask_hbm.at[pl.ds(mbase + c * MSK_C, MSK_C)], mk[s], sm)
        pltpu.async_copy(regr_hbm.at[pl.ds(fbase + c * VAL_C, VAL_C)], rr[s], sr)
        pltpu.async_copy(gt_hbm.at[pl.ds(fbase + c * VAL_C, VAL_C)], gg[s], sg)

    def wait(s):
        sm, sr, sg = sems[s]
        pltpu.make_async_copy(mask_hbm.at[pl.ds(0, MSK_C)], mk[s], sm).wait()
        pltpu.make_async_copy(regr_hbm.at[pl.ds(0, VAL_C)], rr[s], sr).wait()
        pltpu.make_async_copy(gt_hbm.at[pl.ds(0, VAL_C)], gg[s], sg).wait()

    def process(s, c, acc, cnt):
        wait(s)
        mk_s, rr_s, gg_s = mk[s], rr[s], gg[s]

        def pstep(p, carry, mk_s=mk_s, rr_s=rr_s, gg_s=gg_s):
            acc2, cnt2 = carry
            mo = p * B
            fo = p * B * CHN
            for j in range(B // L):
                m = mk_s[pl.ds(mo + j * L, L)].astype(jnp.float32)
                r0 = rr_s[pl.ds(fo + j * L, L)]
                g0 = gg_s[pl.ds(fo + j * L, L)]
                r1 = rr_s[pl.ds(fo + B + j * L, L)]
                g1 = gg_s[pl.ds(fo + B + j * L, L)]
                acc2 = acc2 + (jnp.abs(r0 - g0) + jnp.abs(r1 - g1)) * m
                cnt2 = cnt2 + m
            return acc2, cnt2

        acc, cnt = lax.fori_loop(0, CHUNK_P, pstep, (acc, cnt))

        @pl.when(c + 2 < NCHUNK)
        def _():
            start(s, c + 2)
        return acc, cnt

    zero = jnp.zeros((L,), jnp.float32)
    start(0, 0)
    start(1, 1)

    def pair_body(c2, carry):
        acc, cnt = carry
        acc, cnt = process(0, c2 * 2, acc, cnt)
        acc, cnt = process(1, c2 * 2 + 1, acc, cnt)
        return acc, cnt

    acc, cnt = lax.fori_loop(0, NCHUNK // 2, pair_body, (zero, zero))
    acc, cnt = process(0, NCHUNK - 1, acc, cnt)

    stage_v[pl.ds(0, L)] = acc
    stage_v[pl.ds(L, L)] = cnt
    pltpu.sync_copy(stage_v, part_hbm.at[wid])


def _reduce_body(part_hbm, out_hbm, red_v, res_v):
    cid = lax.axis_index("c")
    sid = lax.axis_index("s")
    wid = cid * NS + sid

    @pl.when(wid == 0)
    def _():
        pltpu.sync_copy(part_hbm, red_v)
        a = jnp.zeros((L,), jnp.float32)
        k = jnp.zeros((L,), jnp.float32)
        for w in range(NW):
            a = a + red_v[w, pl.ds(0, L)]
            k = k + red_v[w, pl.ds(L, L)]
        s_loss = jnp.sum(a)
        num = jnp.sum(k) * 2.0
        res = jnp.full((L,), s_loss, jnp.float32) / (jnp.full((L,), num, jnp.float32) + 1e-4)
        res_v[...] = res
        pltpu.sync_copy(res_v, out_hbm)


@jax.jit
def _masked_l1(regr, gt_regr, mask):
    regr_f = jnp.transpose(regr, (1, 2, 0)).reshape(-1)
    gt_f = jnp.transpose(gt_regr, (1, 2, 0)).reshape(-1)
    mask_f = jnp.transpose(mask, (1, 0)).reshape(-1)
    mesh = plsc.VectorSubcoreMesh(core_axis_name="c", subcore_axis_name="s")
    partials = pl.kernel(
        _partial_body,
        out_type=jax.ShapeDtypeStruct((NW, 2 * L), jnp.float32),
        mesh=mesh,
        compiler_params=pltpu.CompilerParams(needs_layout_passes=False),
        scratch_types=[
            pltpu.VMEM((MSK_C,), jnp.int32),
            pltpu.VMEM((MSK_C,), jnp.int32),
            pltpu.VMEM((VAL_C,), jnp.float32),
            pltpu.VMEM((VAL_C,), jnp.float32),
            pltpu.VMEM((VAL_C,), jnp.float32),
            pltpu.VMEM((VAL_C,), jnp.float32),
            pltpu.VMEM((2 * L,), jnp.float32),
            pltpu.SemaphoreType.DMA,
            pltpu.SemaphoreType.DMA,
            pltpu.SemaphoreType.DMA,
            pltpu.SemaphoreType.DMA,
            pltpu.SemaphoreType.DMA,
            pltpu.SemaphoreType.DMA,
        ],
    )(regr_f, gt_f, mask_f)

    out = pl.kernel(
        _reduce_body,
        out_type=jax.ShapeDtypeStruct((L,), jnp.float32),
        mesh=mesh,
        compiler_params=pltpu.CompilerParams(needs_layout_passes=False),
        scratch_types=[
            pltpu.VMEM((NW, 2 * L), jnp.float32),
            pltpu.VMEM((L,), jnp.float32),
        ],
    )(partials)
    return out


def kernel(regr, gt_regr, mask):
    out = _masked_l1(regr, gt_regr, mask)
    return out[0]


# speedup vs baseline: 124.9260x; 1.0335x over previous
import jax
import jax.numpy as jnp
from jax import lax
from jax.experimental import pallas as pl
from jax.experimental.pallas import tpu as pltpu
from jax.experimental.pallas import tpu_sc as plsc

NC = 2
NS = 16
NW = NC * NS
L = 16

B, P, CHN = 128, 20000, 2
POS_W = P // NW
CHUNK_P = 25
NCHUNK = POS_W // CHUNK_P
MSK_C = CHUNK_P * B
VAL_C = CHUNK_P * B * CHN


def _partial_body(regr_hbm, gt_hbm, mask_hbm, part_hbm,
                  mk0, mk1, rr0, rr1, gg0, gg1, stage_v,
                  sm0, sm1, sr0, sr1, sg0, sg1):
    cid = lax.axis_index("c")
    sid = lax.axis_index("s")
    wid = cid * NS + sid

    mbase = wid * POS_W * B
    fbase = wid * POS_W * B * CHN

    mk = (mk0, mk1)
    rr = (rr0, rr1)
    gg = (gg0, gg1)
    sems = ((sm0, sr0, sg0), (sm1, sr1, sg1))

    def start(s, c):
        sm, sr, sg = sems[s]
        pltpu.async_copy(mask_hbm.at[pl.ds(mbase + c * MSK_C, MSK_C)], mk[s], sm)
        pltpu.async_copy(regr_hbm.at[pl.ds(fbase + c * VAL_C, VAL_C)], rr[s], sr)
        pltpu.async_copy(gt_hbm.at[pl.ds(fbase + c * VAL_C, VAL_C)], gg[s], sg)

    def wait(s):
        sm, sr, sg = sems[s]
        pltpu.make_async_copy(mask_hbm.at[pl.ds(0, MSK_C)], mk[s], sm).wait()
        pltpu.make_async_copy(regr_hbm.at[pl.ds(0, VAL_C)], rr[s], sr).wait()
        pltpu.make_async_copy(gt_hbm.at[pl.ds(0, VAL_C)], gg[s], sg).wait()

    def process(s, c, acc, cnt):
        wait(s)
        mk_s, rr_s, gg_s = mk[s], rr[s], gg[s]

        def pstep(p, carry, mk_s=mk_s, rr_s=rr_s, gg_s=gg_s):
            acc2, cnt2 = carry
            mo = p * B
            fo = p * B * CHN
            for j in range(B // L):
                m = mk_s[pl.ds(mo + j * L, L)].astype(jnp.float32)
                r0 = rr_s[pl.ds(fo + j * L, L)]
                g0 = gg_s[pl.ds(fo + j * L, L)]
                r1 = rr_s[pl.ds(fo + B + j * L, L)]
                g1 = gg_s[pl.ds(fo + B + j * L, L)]
                acc2 = acc2 + (jnp.abs(r0 - g0) + jnp.abs(r1 - g1)) * m
                cnt2 = cnt2 + m
            return acc2, cnt2

        acc, cnt = lax.fori_loop(0, CHUNK_P, pstep, (acc, cnt))

        @pl.when(c + 2 < NCHUNK)
        def _():
            start(s, c + 2)
        return acc, cnt

    zero = jnp.zeros((L,), jnp.float32)
    start(0, 0)
    start(1, 1)

    def pair_body(c2, carry):
        acc, cnt = carry
        acc, cnt = process(0, c2 * 2, acc, cnt)
        acc, cnt = process(1, c2 * 2 + 1, acc, cnt)
        return acc, cnt

    acc, cnt = lax.fori_loop(0, NCHUNK // 2, pair_body, (zero, zero))
    acc, cnt = process(0, NCHUNK - 1, acc, cnt)

    stage_v[pl.ds(0, L)] = acc
    stage_v[pl.ds(L, L)] = cnt
    pltpu.sync_copy(stage_v, part_hbm.at[wid])


def _reduce_body(part_hbm, out_hbm, red_v, res_v):
    cid = lax.axis_index("c")
    sid = lax.axis_index("s")
    wid = cid * NS + sid

    @pl.when(wid == 0)
    def _():
        pltpu.sync_copy(part_hbm, red_v)
        a = jnp.zeros((L,), jnp.float32)
        k = jnp.zeros((L,), jnp.float32)
        for w in range(NW):
            a = a + red_v[w, pl.ds(0, L)]
            k = k + red_v[w, pl.ds(L, L)]
        s_loss = jnp.sum(a)
        num = jnp.sum(k) * 2.0
        res = jnp.full((L,), s_loss, jnp.float32) / (jnp.full((L,), num, jnp.float32) + 1e-4)
        res_v[...] = res
        pltpu.sync_copy(res_v, out_hbm)


@jax.jit
def _masked_l1(regr, gt_regr, mask):
    regr_f = jnp.transpose(regr, (1, 2, 0)).reshape(-1)
    gt_f = jnp.transpose(gt_regr, (1, 2, 0)).reshape(-1)
    mask_f = jnp.transpose(mask, (1, 0)).reshape(-1)
    mesh = plsc.VectorSubcoreMesh(core_axis_name="c", subcore_axis_name="s")
    partials = pl.kernel(
        _partial_body,
        out_type=jax.ShapeDtypeStruct((NW, 2 * L), jnp.float32),
        mesh=mesh,
        compiler_params=pltpu.CompilerParams(needs_layout_passes=False),
        scratch_types=[
            pltpu.VMEM((MSK_C,), jnp.int32),
            pltpu.VMEM((MSK_C,), jnp.int32),
            pltpu.VMEM((VAL_C,), jnp.float32),
            pltpu.VMEM((VAL_C,), jnp.float32),
            pltpu.VMEM((VAL_C,), jnp.float32),
            pltpu.VMEM((VAL_C,), jnp.float32),
            pltpu.VMEM((2 * L,), jnp.float32),
            pltpu.SemaphoreType.DMA,
            pltpu.SemaphoreType.DMA,
            pltpu.SemaphoreType.DMA,
            pltpu.SemaphoreType.DMA,
            pltpu.SemaphoreType.DMA,
            pltpu.SemaphoreType.DMA,
        ],
    )(regr_f, gt_f, mask_f)

    s_loss = jnp.sum(partials[:, :L])
    num = jnp.sum(partials[:, L:]) * 2.0
    return jnp.full((L,), s_loss / (num + 1e-4))


def kernel(regr, gt_regr, mask):
    out = _masked_l1(regr, gt_regr, mask)
    return out[0]


# speedup vs baseline: 126.0849x; 1.0093x over previous
import jax
import jax.numpy as jnp
from jax import lax
from jax.experimental import pallas as pl
from jax.experimental.pallas import tpu as pltpu
from jax.experimental.pallas import tpu_sc as plsc

NC = 2
NS = 16
NW = NC * NS
L = 16

B, P, CHN = 128, 20000, 2

P_SC = 8000
P_TC = P - P_SC
POS_W = P_SC // NW
CHUNK_P = 25
NCHUNK = POS_W // CHUNK_P
MSK_C = CHUNK_P * B
VAL_C = CHUNK_P * B * CHN

PB = 400
NTCB = P_TC // PB
TCOFF = P_SC // PB


def _partial_body(regr_hbm, gt_hbm, mask_hbm, part_hbm,
                  mk0, mk1, rr0, rr1, gg0, gg1, stage_v,
                  sm0, sm1, sr0, sr1, sg0, sg1):
    cid = lax.axis_index("c")
    sid = lax.axis_index("s")
    wid = cid * NS + sid

    mbase = wid * POS_W * B
    fbase = wid * POS_W * B * CHN

    mk = (mk0, mk1)
    rr = (rr0, rr1)
    gg = (gg0, gg1)
    sems = ((sm0, sr0, sg0), (sm1, sr1, sg1))

    def start(s, c):
        sm, sr, sg = sems[s]
        pltpu.async_copy(mask_hbm.at[pl.ds(mbase + c * MSK_C, MSK_C)], mk[s], sm)
        pltpu.async_copy(regr_hbm.at[pl.ds(fbase + c * VAL_C, VAL_C)], rr[s], sr)
        pltpu.async_copy(gt_hbm.at[pl.ds(fbase + c * VAL_C, VAL_C)], gg[s], sg)

    def wait(s):
        sm, sr, sg = sems[s]
        pltpu.make_async_copy(mask_hbm.at[pl.ds(0, MSK_C)], mk[s], sm).wait()
        pltpu.make_async_copy(regr_hbm.at[pl.ds(0, VAL_C)], rr[s], sr).wait()
        pltpu.make_async_copy(gt_hbm.at[pl.ds(0, VAL_C)], gg[s], sg).wait()

    def process(s, c, acc, cnt):
        wait(s)
        mk_s, rr_s, gg_s = mk[s], rr[s], gg[s]

        def pstep(p, carry, mk_s=mk_s, rr_s=rr_s, gg_s=gg_s):
            acc2, cnt2 = carry
            mo = p * B
            fo = p * B * CHN
            for j in range(B // L):
                m = mk_s[pl.ds(mo + j * L, L)].astype(jnp.float32)
                r0 = rr_s[pl.ds(fo + j * L, L)]
                g0 = gg_s[pl.ds(fo + j * L, L)]
                r1 = rr_s[pl.ds(fo + B + j * L, L)]
                g1 = gg_s[pl.ds(fo + B + j * L, L)]
                acc2 = acc2 + (jnp.abs(r0 - g0) + jnp.abs(r1 - g1)) * m
                cnt2 = cnt2 + m
            return acc2, cnt2

        acc, cnt = lax.fori_loop(0, CHUNK_P, pstep, (acc, cnt))

        @pl.when(c + 2 < NCHUNK)
        def _():
            start(s, c + 2)
        return acc, cnt

    zero = jnp.zeros((L,), jnp.float32)
    start(0, 0)
    start(1, 1)

    def pair_body(c2, carry):
        acc, cnt = carry
        acc, cnt = process(0, c2 * 2, acc, cnt)
        acc, cnt = process(1, c2 * 2 + 1, acc, cnt)
        return acc, cnt

    acc, cnt = lax.fori_loop(0, NCHUNK // 2, pair_body, (zero, zero))
    if NCHUNK % 2:
        acc, cnt = process(0, NCHUNK - 1, acc, cnt)

    stage_v[pl.ds(0, L)] = acc
    stage_v[pl.ds(L, L)] = cnt
    pltpu.sync_copy(stage_v, part_hbm.at[wid])


def _tc_body(regr_ref, gt_ref, mask_ref, loss_ref, cnt_ref):
    i = pl.program_id(0)
    m = mask_ref[...].astype(jnp.float32)
    d = jnp.abs(regr_ref[...] - gt_ref[...])
    me = jnp.broadcast_to(m[:, None, :], (PB, 2, B)).reshape(2 * PB, B)
    bl = jnp.sum(d * me, axis=0, keepdims=True)
    bc = jnp.sum(m, axis=0, keepdims=True)

    @pl.when(i == 0)
    def _():
        loss_ref[...] = jnp.zeros((1, B), jnp.float32)
        cnt_ref[...] = jnp.zeros((1, B), jnp.float32)

    loss_ref[...] += bl
    cnt_ref[...] += bc


def _reduce_body(part_hbm, tcl_hbm, tcc_hbm, out_hbm, red_v, tl_v, tc_v, res_v):
    cid = lax.axis_index("c")
    sid = lax.axis_index("s")
    wid = cid * NS + sid

    @pl.when(wid == 0)
    def _():
        pltpu.sync_copy(part_hbm, red_v)
        pltpu.sync_copy(tcl_hbm, tl_v)
        pltpu.sync_copy(tcc_hbm, tc_v)
        a = jnp.zeros((L,), jnp.float32)
        k = jnp.zeros((L,), jnp.float32)
        for w in range(NW):
            a = a + red_v[w, pl.ds(0, L)]
            k = k + red_v[w, pl.ds(L, L)]
        for j in range(B // L):
            a = a + tl_v[0, pl.ds(j * L, L)]
            k = k + tc_v[0, pl.ds(j * L, L)]
        s_loss = jnp.sum(a)
        num = jnp.sum(k) * 2.0
        res = jnp.full((L,), s_loss, jnp.float32) / (jnp.full((L,), num, jnp.float32) + 1e-4)
        res_v[...] = res
        pltpu.sync_copy(res_v, out_hbm)


@jax.jit
def _masked_l1(regr, gt_regr, mask):
    regr_f = jnp.transpose(regr, (1, 2, 0)).reshape(-1)
    gt_f = jnp.transpose(gt_regr, (1, 2, 0)).reshape(-1)
    mask_f = jnp.transpose(mask, (1, 0)).reshape(-1)
    regr_2 = regr_f.reshape(P * CHN, B)
    gt_2 = gt_f.reshape(P * CHN, B)
    mask_2 = mask_f.reshape(P, B)

    mesh = plsc.VectorSubcoreMesh(core_axis_name="c", subcore_axis_name="s")
    partials = pl.kernel(
        _partial_body,
        out_type=jax.ShapeDtypeStruct((NW, 2 * L), jnp.float32),
        mesh=mesh,
        compiler_params=pltpu.CompilerParams(needs_layout_passes=False),
        scratch_types=[
            pltpu.VMEM((MSK_C,), jnp.int32),
            pltpu.VMEM((MSK_C,), jnp.int32),
            pltpu.VMEM((VAL_C,), jnp.float32),
            pltpu.VMEM((VAL_C,), jnp.float32),
            pltpu.VMEM((VAL_C,), jnp.float32),
            pltpu.VMEM((VAL_C,), jnp.float32),
            pltpu.VMEM((2 * L,), jnp.float32),
            pltpu.SemaphoreType.DMA,
            pltpu.SemaphoreType.DMA,
            pltpu.SemaphoreType.DMA,
            pltpu.SemaphoreType.DMA,
            pltpu.SemaphoreType.DMA,
            pltpu.SemaphoreType.DMA,
        ],
    )(regr_f, gt_f, mask_f)

    tc_loss, tc_cnt = pl.pallas_call(
        _tc_body,
        grid=(NTCB,),
        in_specs=[
            pl.BlockSpec((2 * PB, B), lambda i: (TCOFF + i, 0)),
            pl.BlockSpec((2 * PB, B), lambda i: (TCOFF + i, 0)),
            pl.BlockSpec((PB, B), lambda i: (TCOFF + i, 0)),
        ],
        out_specs=[
            pl.BlockSpec((1, B), lambda i: (0, 0)),
            pl.BlockSpec((1, B), lambda i: (0, 0)),
        ],
        out_shape=[
            jax.ShapeDtypeStruct((1, B), jnp.float32),
            jax.ShapeDtypeStruct((1, B), jnp.float32),
        ],
    )(regr_2, gt_2, mask_2)

    out = pl.kernel(
        _reduce_body,
        out_type=jax.ShapeDtypeStruct((L,), jnp.float32),
        mesh=mesh,
        compiler_params=pltpu.CompilerParams(needs_layout_passes=False),
        scratch_types=[
            pltpu.VMEM((NW, 2 * L), jnp.float32),
            pltpu.VMEM((1, B), jnp.float32),
            pltpu.VMEM((1, B), jnp.float32),
            pltpu.VMEM((L,), jnp.float32),
        ],
    )(partials, tc_loss, tc_cnt)
    return out


def kernel(regr, gt_regr, mask):
    out = _masked_l1(regr, gt_regr, mask)
    return out[0]


# speedup vs baseline: 153.6956x; 1.2190x over previous
import jax
import jax.numpy as jnp
from jax import lax
from jax.experimental import pallas as pl
from jax.experimental.pallas import tpu as pltpu
from jax.experimental.pallas import tpu_sc as plsc

NC = 2
NS = 16
NW = NC * NS
L = 16

B, P, CHN = 128, 20000, 2

P_SC = 12000
P_TC = P - P_SC
POS_W = P_SC // NW
CHUNK_P = 25
NCHUNK = POS_W // CHUNK_P
MSK_C = CHUNK_P * B
VAL_C = CHUNK_P * B * CHN

PB = 400
NTCB = P_TC // PB
TCOFF = P_SC // PB


def _partial_body(regr_hbm, gt_hbm, mask_hbm, part_hbm,
                  mk0, mk1, rr0, rr1, gg0, gg1, stage_v,
                  sm0, sm1, sr0, sr1, sg0, sg1):
    cid = lax.axis_index("c")
    sid = lax.axis_index("s")
    wid = cid * NS + sid

    mbase = wid * POS_W * B
    fbase = wid * POS_W * B * CHN

    mk = (mk0, mk1)
    rr = (rr0, rr1)
    gg = (gg0, gg1)
    sems = ((sm0, sr0, sg0), (sm1, sr1, sg1))

    def start(s, c):
        sm, sr, sg = sems[s]
        pltpu.async_copy(mask_hbm.at[pl.ds(mbase + c * MSK_C, MSK_C)], mk[s], sm)
        pltpu.async_copy(regr_hbm.at[pl.ds(fbase + c * VAL_C, VAL_C)], rr[s], sr)
        pltpu.async_copy(gt_hbm.at[pl.ds(fbase + c * VAL_C, VAL_C)], gg[s], sg)

    def wait(s):
        sm, sr, sg = sems[s]
        pltpu.make_async_copy(mask_hbm.at[pl.ds(0, MSK_C)], mk[s], sm).wait()
        pltpu.make_async_copy(regr_hbm.at[pl.ds(0, VAL_C)], rr[s], sr).wait()
        pltpu.make_async_copy(gt_hbm.at[pl.ds(0, VAL_C)], gg[s], sg).wait()

    def process(s, c, acc, cnt):
        wait(s)
        mk_s, rr_s, gg_s = mk[s], rr[s], gg[s]

        def pstep(p, carry, mk_s=mk_s, rr_s=rr_s, gg_s=gg_s):
            acc2, cnt2 = carry
            mo = p * B
            fo = p * B * CHN
            for j in range(B // L):
                m = mk_s[pl.ds(mo + j * L, L)].astype(jnp.float32)
                r0 = rr_s[pl.ds(fo + j * L, L)]
                g0 = gg_s[pl.ds(fo + j * L, L)]
                r1 = rr_s[pl.ds(fo + B + j * L, L)]
                g1 = gg_s[pl.ds(fo + B + j * L, L)]
                acc2 = acc2 + (jnp.abs(r0 - g0) + jnp.abs(r1 - g1)) * m
                cnt2 = cnt2 + m
            return acc2, cnt2

        acc, cnt = lax.fori_loop(0, CHUNK_P, pstep, (acc, cnt))

        @pl.when(c + 2 < NCHUNK)
        def _():
            start(s, c + 2)
        return acc, cnt

    zero = jnp.zeros((L,), jnp.float32)
    start(0, 0)
    start(1, 1)

    def pair_body(c2, carry):
        acc, cnt = carry
        acc, cnt = process(0, c2 * 2, acc, cnt)
        acc, cnt = process(1, c2 * 2 + 1, acc, cnt)
        return acc, cnt

    acc, cnt = lax.fori_loop(0, NCHUNK // 2, pair_body, (zero, zero))
    if NCHUNK % 2:
        acc, cnt = process(0, NCHUNK - 1, acc, cnt)

    stage_v[pl.ds(0, L)] = acc
    stage_v[pl.ds(L, L)] = cnt
    pltpu.sync_copy(stage_v, part_hbm.at[wid])


def _tc_body(regr_ref, gt_ref, mask_ref, loss_ref, cnt_ref):
    i = pl.program_id(0)
    m = mask_ref[...].astype(jnp.float32)
    d = jnp.abs(regr_ref[...] - gt_ref[...])
    me = jnp.broadcast_to(m[:, None, :], (PB, 2, B)).reshape(2 * PB, B)
    bl = jnp.sum(d * me, axis=0, keepdims=True)
    bc = jnp.sum(m, axis=0, keepdims=True)

    @pl.when(i == 0)
    def _():
        loss_ref[...] = jnp.zeros((1, B), jnp.float32)
        cnt_ref[...] = jnp.zeros((1, B), jnp.float32)

    loss_ref[...] += bl
    cnt_ref[...] += bc


def _combine_body(part_ref, tcl_ref, tcc_ref, out_ref):
    s_loss = jnp.sum(part_ref[:, :L]) + jnp.sum(tcl_ref[...])
    s_cnt = jnp.sum(part_ref[:, L:]) + jnp.sum(tcc_ref[...])
    num = s_cnt * 2.0
    out_ref[...] = jnp.full((1, L), s_loss / (num + 1e-4), jnp.float32)


@jax.jit
def _masked_l1(regr, gt_regr, mask):
    regr_f = jnp.transpose(regr, (1, 2, 0)).reshape(-1)
    gt_f = jnp.transpose(gt_regr, (1, 2, 0)).reshape(-1)
    mask_f = jnp.transpose(mask, (1, 0)).reshape(-1)
    regr_2 = regr_f.reshape(P * CHN, B)
    gt_2 = gt_f.reshape(P * CHN, B)
    mask_2 = mask_f.reshape(P, B)

    mesh = plsc.VectorSubcoreMesh(core_axis_name="c", subcore_axis_name="s")
    partials = pl.kernel(
        _partial_body,
        out_type=jax.ShapeDtypeStruct((NW, 2 * L), jnp.float32),
        mesh=mesh,
        compiler_params=pltpu.CompilerParams(needs_layout_passes=False),
        scratch_types=[
            pltpu.VMEM((MSK_C,), jnp.int32),
            pltpu.VMEM((MSK_C,), jnp.int32),
            pltpu.VMEM((VAL_C,), jnp.float32),
            pltpu.VMEM((VAL_C,), jnp.float32),
            pltpu.VMEM((VAL_C,), jnp.float32),
            pltpu.VMEM((VAL_C,), jnp.float32),
            pltpu.VMEM((2 * L,), jnp.float32),
            pltpu.SemaphoreType.DMA,
            pltpu.SemaphoreType.DMA,
            pltpu.SemaphoreType.DMA,
            pltpu.SemaphoreType.DMA,
            pltpu.SemaphoreType.DMA,
            pltpu.SemaphoreType.DMA,
        ],
    )(regr_f, gt_f, mask_f)

    tc_loss, tc_cnt = pl.pallas_call(
        _tc_body,
        grid=(NTCB,),
        in_specs=[
            pl.BlockSpec((2 * PB, B), lambda i: (TCOFF + i, 0)),
            pl.BlockSpec((2 * PB, B), lambda i: (TCOFF + i, 0)),
            pl.BlockSpec((PB, B), lambda i: (TCOFF + i, 0)),
        ],
        out_specs=[
            pl.BlockSpec((1, B), lambda i: (0, 0)),
            pl.BlockSpec((1, B), lambda i: (0, 0)),
        ],
        out_shape=[
            jax.ShapeDtypeStruct((1, B), jnp.float32),
            jax.ShapeDtypeStruct((1, B), jnp.float32),
        ],
    )(regr_2, gt_2, mask_2)

    out = pl.pallas_call(
        _combine_body,
        out_shape=jax.ShapeDtypeStruct((1, L), jnp.float32),
    )(partials, tc_loss, tc_cnt)
    return out


def kernel(regr, gt_regr, mask):
    out = _masked_l1(regr, gt_regr, mask)
    return out[0, 0]
